# baseline (speedup 1.0000x reference)
import functools

import jax
import jax.numpy as jnp
from jax import lax
from jax.experimental import pallas as pl
from jax.experimental.pallas import tpu as pltpu
from jax.experimental.pallas import tpu_sc as plsc

N = 10000
E = 320000
HV, HE, P, Q = 128, 64, 16, 16
D = HV + P + Q

NC, NS = 2, 16
NW = NC * NS
GW = 128
SW = 128

K = 4
C = E // K
CPAD = 2 * C + (-2 * C) % (GW * NW)

EB = 2000
VB = 2000


def _sc_gather(table, idx_flat):
    n_idx = idx_flat.shape[0]
    idx2d = idx_flat.reshape(1, n_idx)
    mesh = plsc.VectorSubcoreMesh(core_axis_name="c", subcore_axis_name="s")

    @functools.partial(
        pl.kernel,
        out_type=jax.ShapeDtypeStruct((n_idx, D), table.dtype),
        mesh=mesh,
        compiler_params=pltpu.CompilerParams(use_tc_tiling_on_sc=False),
    )
    def k(table_hbm, idx_hbm, out_hbm):
        def body(i_vmem, o_vmem):
            pltpu.sync_copy(table_hbm.at[i_vmem.at[0]], o_vmem)

        pltpu.emit_pipeline(
            body,
            grid=(n_idx // GW,),
            in_specs=[pl.BlockSpec((1, GW), lambda i: (0, i))],
            out_specs=[pl.BlockSpec((GW, D), lambda i: (i, 0))],
            core_axis_name=("c", "s"),
            dimension_semantics=(pltpu.PARALLEL,),
        )(idx_hbm, out_hbm)

    return k(table, idx2d)


def _sc_segment_sum(msg, dst3d):
    n_rows = dst3d.shape[0]
    n_full = N // SW
    tail = N - n_full * SW
    n_chunks = n_full + (1 if tail else 0)
    chunk_iters = (n_chunks + NS - 1) // NS
    mesh = plsc.VectorSubcoreMesh(core_axis_name="c", subcore_axis_name="s")
    max_iters = (n_rows + NW - 1) // NW

    @functools.partial(
        pl.kernel,
        out_type=jax.ShapeDtypeStruct((NC, N, HV), jnp.float32),
        mesh=mesh,
        scratch_types=[
            pltpu.VMEM((1, SW), jnp.int32),
            pltpu.VMEM((SW, HV), jnp.float32),
            pltpu.VMEM_SHARED((N, HV), jnp.float32),
        ],
    )
    def k(msg_hbm, dst_hbm, out_hbm, idx_v, buf_v, acc_sh):
        cid = lax.axis_index("c")
        sid = lax.axis_index("s")
        wid = sid * NC + cid

        @pl.loop(0, SW)
        def _(r):
            @pl.loop(0, HV, step=16)
            def _(col):
                buf_v[r, pl.ds(col, 16)] = jnp.zeros((16,), jnp.float32)

        @pl.loop(0, chunk_iters)
        def _(j):
            c = sid + j * NS

            @pl.when(c < n_full)
            def _():
                pltpu.sync_copy(buf_v, acc_sh.at[pl.ds(c * SW, SW)])

            @pl.when(c == n_full)
            def _():
                pltpu.sync_copy(buf_v.at[pl.ds(0, tail)],
                                acc_sh.at[pl.ds(n_full * SW, tail)])

        plsc.subcore_barrier()

        @pl.loop(0, max_iters)
        def _(j):
            r = wid + j * NW

            @pl.when(r < n_rows)
            def _():
                pltpu.sync_copy(dst_hbm.at[r], idx_v)
                pltpu.sync_copy(msg_hbm.at[pl.ds(r * SW, SW)], buf_v)
                pltpu.sync_copy(buf_v, acc_sh.at[idx_v.at[0]], add=True)

        plsc.subcore_barrier()

        @pl.loop(0, chunk_iters)
        def _(j):
            c = sid + j * NS

            @pl.when(c < n_full)
            def _():
                pltpu.sync_copy(acc_sh.at[pl.ds(c * SW, SW)],
                                out_hbm.at[cid, pl.ds(c * SW, SW)])

            @pl.when(c == n_full)
            def _():
                pltpu.sync_copy(acc_sh.at[pl.ds(n_full * SW, tail)],
                                out_hbm.at[cid, pl.ds(n_full * SW, tail)])

    return k(msg, dst3d)


def _edge_body(xs_ref, xd_ref, he_ref,
               wm_hv_ref, wm_pq_ref, wm_he_ref, bm_ref,
               we_u_ref, we_v_ref, we_e_ref, be_ref,
               wih_ref, whh_ref, bih_ref, bhh_ref,
               msg_ref, heo_ref):
    xs = xs_ref[...]
    xd = xd_ref[...]
    he = he_ref[...]
    hv_u = xs[:, :HV]
    hv_v = xd[:, :HV]
    dpq = xs[:, HV:] - xd[:, HV:]

    msg = (jnp.dot(hv_u, wm_hv_ref[...], preferred_element_type=jnp.float32)
           + jnp.dot(dpq, wm_pq_ref[...], preferred_element_type=jnp.float32)
           + jnp.dot(he, wm_he_ref[...], preferred_element_type=jnp.float32)
           + bm_ref[...])
    msg_ref[...] = jnp.maximum(msg, 0.0)

    me = (jnp.dot(hv_u, we_u_ref[...], preferred_element_type=jnp.float32)
          + jnp.dot(hv_v, we_v_ref[...], preferred_element_type=jnp.float32)
          + jnp.dot(he, we_e_ref[...], preferred_element_type=jnp.float32)
          + be_ref[...])
    me = jnp.maximum(me, 0.0)

    gi = jnp.dot(me, wih_ref[...], preferred_element_type=jnp.float32) + bih_ref[...]
    gh = jnp.dot(he, whh_ref[...], preferred_element_type=jnp.float32) + bhh_ref[...]
    r = jax.nn.sigmoid(gi[:, :HE] + gh[:, :HE])
    z = jax.nn.sigmoid(gi[:, HE:2 * HE] + gh[:, HE:2 * HE])
    n = jnp.tanh(gi[:, 2 * HE:] + r * gh[:, 2 * HE:])
    heo_ref[...] = (1.0 - z) * n + z * he


def _tc_edge(xs_xd, he_full, k, wm_hv, wm_pq, wm_he, bm2,
             we_u, we_v, we_e, be2, wih_e_t, whh_e_t, bih_e2, bhh_e2):
    nsteps = C // EB
    off = k * nsteps
    full = lambda arr: pl.BlockSpec(arr.shape, lambda i: (0,) * arr.ndim)
    return pl.pallas_call(
        _edge_body,
        grid=(nsteps,),
        in_specs=[
            pl.BlockSpec((EB, D), lambda i: (i, 0)),
            pl.BlockSpec((EB, D), lambda i: (i + nsteps, 0)),
            pl.BlockSpec((EB, HE), lambda i, o=off: (i + o, 0)),
            full(wm_hv), full(wm_pq), full(wm_he), full(bm2),
            full(we_u), full(we_v), full(we_e), full(be2),
            full(wih_e_t), full(whh_e_t), full(bih_e2), full(bhh_e2),
        ],
        out_specs=[
            pl.BlockSpec((EB, HV), lambda i: (i, 0)),
            pl.BlockSpec((EB, HE), lambda i: (i, 0)),
        ],
        out_shape=[
            jax.ShapeDtypeStruct((C, HV), jnp.float32),
            jax.ShapeDtypeStruct((C, HE), jnp.float32),
        ],
    )(xs_xd, xs_xd, he_full, wm_hv, wm_pq, wm_he, bm2,
      we_u, we_v, we_e, be2, wih_e_t, whh_e_t, bih_e2, bhh_e2)


def _vertex_body(p0_ref, p1_ref, p2_ref, p3_ref,
                 hv_ref, wih_ref, whh_ref, bih_ref, bhh_ref, out_ref):
    mv = (p0_ref[0] + p0_ref[1] + p1_ref[0] + p1_ref[1]
          + p2_ref[0] + p2_ref[1] + p3_ref[0] + p3_ref[1])
    hv = hv_ref[...]
    gi = jnp.dot(mv, wih_ref[...], preferred_element_type=jnp.float32) + bih_ref[...]
    gh = jnp.dot(hv, whh_ref[...], preferred_element_type=jnp.float32) + bhh_ref[...]
    r = jax.nn.sigmoid(gi[:, :HV] + gh[:, :HV])
    z = jax.nn.sigmoid(gi[:, HV:2 * HV] + gh[:, HV:2 * HV])
    n = jnp.tanh(gi[:, 2 * HV:] + r * gh[:, 2 * HV:])
    out_ref[...] = (1.0 - z) * n + z * hv


def _tc_vertex(parts_list, hv_ftr, wih_v_t, whh_v_t, bih_v2, bhh_v2):
    nsteps = N // VB
    full = lambda arr: pl.BlockSpec(arr.shape, lambda i: (0,) * arr.ndim)
    pspec = pl.BlockSpec((NC, VB, HV), lambda i: (0, i, 0))
    return pl.pallas_call(
        _vertex_body,
        grid=(nsteps,),
        in_specs=[
            pspec, pspec, pspec, pspec,
            pl.BlockSpec((VB, HV), lambda i: (i, 0)),
            full(wih_v_t), full(whh_v_t), full(bih_v2), full(bhh_v2),
        ],
        out_specs=pl.BlockSpec((VB, HV), lambda i: (i, 0)),
        out_shape=jax.ShapeDtypeStruct((N, HV), jnp.float32),
    )(*parts_list, hv_ftr, wih_v_t, whh_v_t, bih_v2, bhh_v2)


def kernel(hv_ftr, he_ftr, p_ftr, q_ftr, Wm, bm, We, be,
           Wih_v, Whh_v, bih_v, bhh_v, Wih_e, Whh_e, bih_e, bhh_e,
           edge_index):
    table = jnp.concatenate([hv_ftr, p_ftr, q_ftr], axis=1)
    src, dst = edge_index[0], edge_index[1]

    wm_hv = Wm[:, :HV].T
    wm_he = Wm[:, HV:HV + HE].T
    wm_pq = Wm[:, HV + HE:].T
    we_u = We[:, :HV].T
    we_v = We[:, HV:2 * HV].T
    we_e = We[:, 2 * HV:].T

    heos, parts_list = [], []
    for k in range(K):
        sl = slice(k * C, (k + 1) * C)
        idx = jnp.pad(jnp.concatenate([src[sl], dst[sl]]), (0, CPAD - 2 * C))
        x = _sc_gather(table, idx)
        msg_k, heo_k = _tc_edge(
            x, he_ftr, k, wm_hv, wm_pq, wm_he, bm.reshape(1, -1),
            we_u, we_v, we_e, be.reshape(1, -1),
            Wih_e.T, Whh_e.T,
            bih_e.reshape(1, -1), bhh_e.reshape(1, -1))
        parts_k = _sc_segment_sum(msg_k, dst[sl].reshape(C // SW, 1, SW))
        heos.append(heo_k)
        parts_list.append(parts_k)

    he_out = jnp.concatenate(heos, axis=0)
    hv_out = _tc_vertex(parts_list, hv_ftr, Wih_v.T, Whh_v.T,
                        bih_v.reshape(1, -1), bhh_v.reshape(1, -1))
    return (hv_out, he_out)


# speedup vs baseline: 1.2155x; 1.2155x over previous
import functools

import jax
import jax.numpy as jnp
from jax import lax
from jax.experimental import pallas as pl
from jax.experimental.pallas import tpu as pltpu
from jax.experimental.pallas import tpu_sc as plsc

N = 10000
E = 320000
HV, HE, P, Q = 128, 64, 16, 16
D = HV + P + Q

NC, NS = 2, 16
NW = NC * NS
GW = 128
NIDX = 2 * E + (-2 * E) % (GW * NW)
SW = 128

EB = 4000
VB = 2000


def _sc_gather(table, idx_flat):
    n_idx = idx_flat.shape[0]
    idx2d = idx_flat.reshape(1, n_idx)
    mesh = plsc.VectorSubcoreMesh(core_axis_name="c", subcore_axis_name="s")

    @functools.partial(
        pl.kernel,
        out_type=jax.ShapeDtypeStruct((n_idx, D), table.dtype),
        mesh=mesh,
        compiler_params=pltpu.CompilerParams(use_tc_tiling_on_sc=False),
    )
    def k(table_hbm, idx_hbm, out_hbm):
        def body(i_vmem, o_vmem):
            pltpu.sync_copy(table_hbm.at[i_vmem.at[0]], o_vmem)

        pltpu.emit_pipeline(
            body,
            grid=(n_idx // GW,),
            in_specs=[pl.BlockSpec((1, GW), lambda i: (0, i))],
            out_specs=[pl.BlockSpec((GW, D), lambda i: (i, 0))],
            core_axis_name=("c", "s"),
            dimension_semantics=(pltpu.PARALLEL,),
        )(idx_hbm, out_hbm)

    return k(table, idx2d)


def _sc_segment_sum(msg, dst3d):
    n_rows = dst3d.shape[0]
    n_full = N // SW
    tail = N - n_full * SW
    n_chunks = n_full + (1 if tail else 0)
    chunk_iters = (n_chunks + NS - 1) // NS
    mesh = plsc.VectorSubcoreMesh(core_axis_name="c", subcore_axis_name="s")
    max_iters = (n_rows + NW - 1) // NW

    @functools.partial(
        pl.kernel,
        out_type=jax.ShapeDtypeStruct((NC, N, HV), jnp.float32),
        mesh=mesh,
        scratch_types=[
            pltpu.VMEM((1, SW), jnp.int32),
            pltpu.VMEM((SW, HV), jnp.float32),
            pltpu.VMEM_SHARED((N, HV), jnp.float32),
        ],
    )
    def k(msg_hbm, dst_hbm, out_hbm, idx_v, buf_v, acc_sh):
        cid = lax.axis_index("c")
        sid = lax.axis_index("s")
        wid = sid * NC + cid

        @pl.loop(0, SW)
        def _(r):
            @pl.loop(0, HV, step=16)
            def _(col):
                buf_v[r, pl.ds(col, 16)] = jnp.zeros((16,), jnp.float32)

        @pl.loop(0, chunk_iters)
        def _(j):
            c = sid + j * NS

            @pl.when(c < n_full)
            def _():
                pltpu.sync_copy(buf_v, acc_sh.at[pl.ds(c * SW, SW)])

            @pl.when(c == n_full)
            def _():
                pltpu.sync_copy(buf_v.at[pl.ds(0, tail)],
                                acc_sh.at[pl.ds(n_full * SW, tail)])

        plsc.subcore_barrier()

        @pl.loop(0, max_iters)
        def _(j):
            r = wid + j * NW

            @pl.when(r < n_rows)
            def _():
                pltpu.sync_copy(dst_hbm.at[r], idx_v)
                pltpu.sync_copy(msg_hbm.at[pl.ds(r * SW, SW)], buf_v)
                pltpu.sync_copy(buf_v, acc_sh.at[idx_v.at[0]], add=True)

        plsc.subcore_barrier()

        @pl.loop(0, chunk_iters)
        def _(j):
            c = sid + j * NS

            @pl.when(c < n_full)
            def _():
                pltpu.sync_copy(acc_sh.at[pl.ds(c * SW, SW)],
                                out_hbm.at[cid, pl.ds(c * SW, SW)])

            @pl.when(c == n_full)
            def _():
                pltpu.sync_copy(acc_sh.at[pl.ds(n_full * SW, tail)],
                                out_hbm.at[cid, pl.ds(n_full * SW, tail)])

    return k(msg, dst3d)


def _edge_body(xs_ref, xd_ref, he_ref,
               wm_hv_ref, wm_pq_ref, wm_he_ref, bm_ref,
               we_u_ref, we_v_ref, we_e_ref, be_ref,
               wih_ref, whh_ref, bih_ref, bhh_ref,
               msg_ref, heo_ref):
    xs = xs_ref[...]
    xd = xd_ref[...]
    he = he_ref[...]
    hv_u = xs[:, :HV]
    hv_v = xd[:, :HV]
    dpq = xs[:, HV:] - xd[:, HV:]

    msg = (jnp.dot(hv_u, wm_hv_ref[...], preferred_element_type=jnp.float32)
           + jnp.dot(dpq, wm_pq_ref[...], preferred_element_type=jnp.float32)
           + jnp.dot(he, wm_he_ref[...], preferred_element_type=jnp.float32)
           + bm_ref[...])
    msg_ref[...] = jnp.maximum(msg, 0.0)

    me = (jnp.dot(hv_u, we_u_ref[...], preferred_element_type=jnp.float32)
          + jnp.dot(hv_v, we_v_ref[...], preferred_element_type=jnp.float32)
          + jnp.dot(he, we_e_ref[...], preferred_element_type=jnp.float32)
          + be_ref[...])
    me = jnp.maximum(me, 0.0)

    gi = jnp.dot(me, wih_ref[...], preferred_element_type=jnp.float32) + bih_ref[...]
    gh = jnp.dot(he, whh_ref[...], preferred_element_type=jnp.float32) + bhh_ref[...]
    r = jax.nn.sigmoid(gi[:, :HE] + gh[:, :HE])
    z = jax.nn.sigmoid(gi[:, HE:2 * HE] + gh[:, HE:2 * HE])
    n = jnp.tanh(gi[:, 2 * HE:] + r * gh[:, 2 * HE:])
    heo_ref[...] = (1.0 - z) * n + z * he


def _tc_edge(xs_xd, he_ftr, wm_hv, wm_pq, wm_he, bm2,
             we_u, we_v, we_e, be2, wih_e_t, whh_e_t, bih_e2, bhh_e2):
    nsteps = E // EB
    full = lambda arr: pl.BlockSpec(arr.shape, lambda i: (0,) * arr.ndim)
    return pl.pallas_call(
        _edge_body,
        grid=(nsteps,),
        in_specs=[
            pl.BlockSpec((EB, D), lambda i: (i, 0)),
            pl.BlockSpec((EB, D), lambda i: (i + nsteps, 0)),
            pl.BlockSpec((EB, HE), lambda i: (i, 0)),
            full(wm_hv), full(wm_pq), full(wm_he), full(bm2),
            full(we_u), full(we_v), full(we_e), full(be2),
            full(wih_e_t), full(whh_e_t), full(bih_e2), full(bhh_e2),
        ],
        out_specs=[
            pl.BlockSpec((EB, HV), lambda i: (i, 0)),
            pl.BlockSpec((EB, HE), lambda i: (i, 0)),
        ],
        out_shape=[
            jax.ShapeDtypeStruct((E, HV), jnp.float32),
            jax.ShapeDtypeStruct((E, HE), jnp.float32),
        ],
    )(xs_xd, xs_xd, he_ftr, wm_hv, wm_pq, wm_he, bm2,
      we_u, we_v, we_e, be2, wih_e_t, whh_e_t, bih_e2, bhh_e2)


def _vertex_body(p0_ref, p1_ref, hv_ref, wih_ref, whh_ref, bih_ref, bhh_ref,
                 out_ref):
    mv = p0_ref[0] + p1_ref[0]
    hv = hv_ref[...]
    gi = jnp.dot(mv, wih_ref[...], preferred_element_type=jnp.float32) + bih_ref[...]
    gh = jnp.dot(hv, whh_ref[...], preferred_element_type=jnp.float32) + bhh_ref[...]
    r = jax.nn.sigmoid(gi[:, :HV] + gh[:, :HV])
    z = jax.nn.sigmoid(gi[:, HV:2 * HV] + gh[:, HV:2 * HV])
    n = jnp.tanh(gi[:, 2 * HV:] + r * gh[:, 2 * HV:])
    out_ref[...] = (1.0 - z) * n + z * hv


def _tc_vertex(parts, hv_ftr, wih_v_t, whh_v_t, bih_v2, bhh_v2):
    nsteps = N // VB
    full = lambda arr: pl.BlockSpec(arr.shape, lambda i: (0,) * arr.ndim)
    return pl.pallas_call(
        _vertex_body,
        grid=(nsteps,),
        in_specs=[
            pl.BlockSpec((1, VB, HV), lambda i: (0, i, 0)),
            pl.BlockSpec((1, VB, HV), lambda i: (0, i, 0)),
            pl.BlockSpec((VB, HV), lambda i: (i, 0)),
            full(wih_v_t), full(whh_v_t), full(bih_v2), full(bhh_v2),
        ],
        out_specs=pl.BlockSpec((VB, HV), lambda i: (i, 0)),
        out_shape=jax.ShapeDtypeStruct((N, HV), jnp.float32),
    )(parts[:1], parts[1:], hv_ftr, wih_v_t, whh_v_t, bih_v2, bhh_v2)


def kernel(hv_ftr, he_ftr, p_ftr, q_ftr, Wm, bm, We, be,
           Wih_v, Whh_v, bih_v, bhh_v, Wih_e, Whh_e, bih_e, bhh_e,
           edge_index):
    table = jnp.concatenate([hv_ftr, p_ftr, q_ftr], axis=1)
    idx_flat = jnp.pad(edge_index.reshape(2 * E), (0, NIDX - 2 * E))
    xs_xd = _sc_gather(table, idx_flat)

    wm_hv = Wm[:, :HV].T
    wm_he = Wm[:, HV:HV + HE].T
    wm_pq = Wm[:, HV + HE:].T
    we_u = We[:, :HV].T
    we_v = We[:, HV:2 * HV].T
    we_e = We[:, 2 * HV:].T
    msg, he_out = _tc_edge(
        xs_xd, he_ftr, wm_hv, wm_pq, wm_he, bm.reshape(1, -1),
        we_u, we_v, we_e, be.reshape(1, -1),
        Wih_e.T, Whh_e.T,
        bih_e.reshape(1, -1), bhh_e.reshape(1, -1))

    dst3d = edge_index[1].reshape(E // SW, 1, SW)
    parts = _sc_segment_sum(msg, dst3d)

    hv_out = _tc_vertex(parts, hv_ftr, Wih_v.T, Whh_v.T,
                        bih_v.reshape(1, -1), bhh_v.reshape(1, -1))
    return (hv_out, he_out)
